# baseline (device time: 70661 ns/iter reference)
import jax
import jax.numpy as jnp
from jax import lax
from jax.experimental import pallas as pl
from jax.experimental.pallas import tpu as pltpu

N_DEV = 16
M, N = 2048, 1024
CH = M // N_DEV
SUB = 4
CS = CH // SUB
STEPS = 7
MESH = pl.DeviceIdType.MESH


def kernel(x):
    def body(x_ref, out_ref, sL, sR, rL, rR, sD, rD,
             rs_sL_sem, rs_rL_sem, rs_sR_sem, rs_rR_sem,
             ag_sL_sem, ag_rL_sem, ag_sR_sem, ag_rR_sem,
             rs_sD_sem, rs_rD_sem, ag_sD_sem, ag_rD_sem):
        p = lax.axis_index("i")
        left = (p + N_DEV - 1) % N_DEV
        right = (p + 1) % N_DEV
        anti = (p + 8) % N_DEV

        def xchunk(c, j):
            sl = pl.ds(c * CH + j * CS, CS)
            return x_ref[0, sl, :].astype(jnp.bfloat16)

        barrier = pltpu.get_barrier_semaphore()
        for nbr in (left, right, anti):
            pl.semaphore_signal(
                barrier, inc=1, device_id=(nbr,), device_id_type=MESH,
            )
        pl.semaphore_wait(barrier, 3)

        sD[:, :] = x_ref[0, pl.ds(anti * CH, CH), :].astype(jnp.bfloat16)
        rs_direct = pltpu.make_async_remote_copy(
            src_ref=sD, dst_ref=rD,
            send_sem=rs_sD_sem, recv_sem=rs_rD_sem,
            device_id=(anti,), device_id_type=MESH,
        )
        rs_direct.start()

        rsL = [None] * (STEPS * SUB)
        rsR = [None] * (STEPS * SUB)
        for s in range(STEPS):
            cL = (p + 9 + s) % N_DEV
            cR = (p + 7 - s) % N_DEV
            for j in range(SUB):
                rows = slice(j * CS, (j + 1) * CS)
                k = s * SUB + j
                if s > 0:
                    rsL[k - SUB].wait()
                    sL[s, rows, :] = xchunk(cL, j) + rL[s - 1, rows, :]
                else:
                    sL[s, rows, :] = xchunk(cL, j)
                rdma = pltpu.make_async_remote_copy(
                    src_ref=sL.at[s, rows],
                    dst_ref=rL.at[s, rows],
                    send_sem=rs_sL_sem.at[k],
                    recv_sem=rs_rL_sem.at[k],
                    device_id=(left,), device_id_type=MESH,
                )
                rdma.start()
                rsL[k] = rdma
                if s > 0:
                    rsR[k - SUB].wait()
                    sR[s, rows, :] = xchunk(cR, j) + rR[s - 1, rows, :]
                else:
                    sR[s, rows, :] = xchunk(cR, j)
                rdma = pltpu.make_async_remote_copy(
                    src_ref=sR.at[s, rows],
                    dst_ref=rR.at[s, rows],
                    send_sem=rs_sR_sem.at[k],
                    recv_sem=rs_rR_sem.at[k],
                    device_id=(right,), device_id_type=MESH,
                )
                rdma.start()
                rsR[k] = rdma

        agL = [None] * (STEPS * SUB)
        agR = [None] * (STEPS * SUB)
        for t in range(STEPS):
            cL = (p + t) % N_DEV
            cR = (p + N_DEV - t) % N_DEV
            for j in range(SUB):
                rows = slice(j * CS, (j + 1) * CS)
                k = t * SUB + j
                if t > 0:
                    agL[k - SUB].wait()
                    agR[k - SUB].wait()
                else:
                    if j == 0:
                        rs_direct.wait()
                    rsL[(STEPS - 1) * SUB + j].wait()
                    rsR[(STEPS - 1) * SUB + j].wait()
                    out_ref[p, rows, :] = (
                        xchunk(p, j)
                        + rL[STEPS - 1, rows, :]
                        + rR[STEPS - 1, rows, :]
                        + rD[rows, :]
                    )
                rdma = pltpu.make_async_remote_copy(
                    src_ref=out_ref.at[cL, rows],
                    dst_ref=out_ref.at[cL, rows],
                    send_sem=ag_sL_sem.at[k],
                    recv_sem=ag_rL_sem.at[k],
                    device_id=(left,), device_id_type=MESH,
                )
                rdma.start()
                agL[k] = rdma
                rdma = pltpu.make_async_remote_copy(
                    src_ref=out_ref.at[cR, rows],
                    dst_ref=out_ref.at[cR, rows],
                    send_sem=ag_sR_sem.at[k],
                    recv_sem=ag_rR_sem.at[k],
                    device_id=(right,), device_id_type=MESH,
                )
                rdma.start()
                agR[k] = rdma
            if t == 0:
                ag_direct = pltpu.make_async_remote_copy(
                    src_ref=out_ref.at[p],
                    dst_ref=out_ref.at[p],
                    send_sem=ag_sD_sem, recv_sem=ag_rD_sem,
                    device_id=(anti,), device_id_type=MESH,
                )
                ag_direct.start()

        for j in range(SUB):
            agL[(STEPS - 1) * SUB + j].wait()
            agR[(STEPS - 1) * SUB + j].wait()
        ag_direct.wait()

    out = pl.pallas_call(
        body,
        out_shape=jax.ShapeDtypeStruct((N_DEV, CH, N), jnp.bfloat16),
        in_specs=[pl.BlockSpec(memory_space=pltpu.VMEM)],
        out_specs=pl.BlockSpec(memory_space=pltpu.VMEM),
        scratch_shapes=[
            pltpu.VMEM((STEPS, CH, N), jnp.bfloat16),
            pltpu.VMEM((STEPS, CH, N), jnp.bfloat16),
            pltpu.VMEM((STEPS, CH, N), jnp.bfloat16),
            pltpu.VMEM((STEPS, CH, N), jnp.bfloat16),
            pltpu.VMEM((CH, N), jnp.bfloat16),
            pltpu.VMEM((CH, N), jnp.bfloat16),
            pltpu.SemaphoreType.DMA((STEPS * SUB,)),
            pltpu.SemaphoreType.DMA((STEPS * SUB,)),
            pltpu.SemaphoreType.DMA((STEPS * SUB,)),
            pltpu.SemaphoreType.DMA((STEPS * SUB,)),
            pltpu.SemaphoreType.DMA((STEPS * SUB,)),
            pltpu.SemaphoreType.DMA((STEPS * SUB,)),
            pltpu.SemaphoreType.DMA((STEPS * SUB,)),
            pltpu.SemaphoreType.DMA((STEPS * SUB,)),
            pltpu.SemaphoreType.DMA(()),
            pltpu.SemaphoreType.DMA(()),
            pltpu.SemaphoreType.DMA(()),
            pltpu.SemaphoreType.DMA(()),
        ],
        compiler_params=pltpu.CompilerParams(collective_id=0),
    )(x)
    return out.reshape(M, N)


# device time: 70628 ns/iter; 1.0005x vs baseline; 1.0005x over previous
import jax
import jax.numpy as jnp
from jax import lax
from jax.experimental import pallas as pl
from jax.experimental.pallas import tpu as pltpu

N_DEV = 16
M, N = 2048, 1024
CH = M // N_DEV
SUB = 4
CS = CH // SUB
STEPS = 7
MESH = pl.DeviceIdType.MESH


def kernel(x):
    def body(x_ref, out_ref, sL, sR, rL, rR, sD, rD,
             rs_sL_sem, rs_rL_sem, rs_sR_sem, rs_rR_sem,
             ag_sL_sem, ag_rL_sem, ag_sR_sem, ag_rR_sem,
             rs_sD_sem, rs_rD_sem, ag_sD_sem, ag_rD_sem):
        p = lax.axis_index("i")
        left = (p + N_DEV - 1) % N_DEV
        right = (p + 1) % N_DEV
        anti = (p + 8) % N_DEV

        def xchunk(c, j):
            sl = pl.ds(c * CH + j * CS, CS)
            return x_ref[0, sl, :].astype(jnp.bfloat16)

        barrier = pltpu.get_barrier_semaphore()
        for nbr in (left, right, anti):
            pl.semaphore_signal(
                barrier, inc=1, device_id=(nbr,), device_id_type=MESH,
            )
        pl.semaphore_wait(barrier, 3)

        rsL = [None] * (STEPS * SUB)
        rsR = [None] * (STEPS * SUB)
        for s in range(STEPS):
            cL = (p + 9 + s) % N_DEV
            cR = (p + 7 - s) % N_DEV
            for j in range(SUB):
                rows = slice(j * CS, (j + 1) * CS)
                k = s * SUB + j
                if s > 0:
                    rsL[k - SUB].wait()
                    sL[s, rows, :] = xchunk(cL, j) + rL[s - 1, rows, :]
                else:
                    sL[s, rows, :] = xchunk(cL, j)
                rdma = pltpu.make_async_remote_copy(
                    src_ref=sL.at[s, rows],
                    dst_ref=rL.at[s, rows],
                    send_sem=rs_sL_sem.at[k],
                    recv_sem=rs_rL_sem.at[k],
                    device_id=(left,), device_id_type=MESH,
                )
                rdma.start()
                rsL[k] = rdma
                if s > 0:
                    rsR[k - SUB].wait()
                    sR[s, rows, :] = xchunk(cR, j) + rR[s - 1, rows, :]
                else:
                    sR[s, rows, :] = xchunk(cR, j)
                rdma = pltpu.make_async_remote_copy(
                    src_ref=sR.at[s, rows],
                    dst_ref=rR.at[s, rows],
                    send_sem=rs_sR_sem.at[k],
                    recv_sem=rs_rR_sem.at[k],
                    device_id=(right,), device_id_type=MESH,
                )
                rdma.start()
                rsR[k] = rdma
            if s == 0:
                sD[:, :] = x_ref[0, pl.ds(anti * CH, CH), :].astype(
                    jnp.bfloat16
                )
                rs_direct = pltpu.make_async_remote_copy(
                    src_ref=sD, dst_ref=rD,
                    send_sem=rs_sD_sem, recv_sem=rs_rD_sem,
                    device_id=(anti,), device_id_type=MESH,
                )
                rs_direct.start()

        agL = [None] * (STEPS * SUB)
        agR = [None] * (STEPS * SUB)
        for t in range(STEPS):
            cL = (p + t) % N_DEV
            cR = (p + N_DEV - t) % N_DEV
            for j in range(SUB):
                rows = slice(j * CS, (j + 1) * CS)
                k = t * SUB + j
                if t > 0:
                    agL[k - SUB].wait()
                    agR[k - SUB].wait()
                else:
                    if j == 0:
                        rs_direct.wait()
                    rsL[(STEPS - 1) * SUB + j].wait()
                    rsR[(STEPS - 1) * SUB + j].wait()
                    out_ref[p, rows, :] = (
                        xchunk(p, j)
                        + rL[STEPS - 1, rows, :]
                        + rR[STEPS - 1, rows, :]
                        + rD[rows, :]
                    )
                rdma = pltpu.make_async_remote_copy(
                    src_ref=out_ref.at[cL, rows],
                    dst_ref=out_ref.at[cL, rows],
                    send_sem=ag_sL_sem.at[k],
                    recv_sem=ag_rL_sem.at[k],
                    device_id=(left,), device_id_type=MESH,
                )
                rdma.start()
                agL[k] = rdma
                rdma = pltpu.make_async_remote_copy(
                    src_ref=out_ref.at[cR, rows],
                    dst_ref=out_ref.at[cR, rows],
                    send_sem=ag_sR_sem.at[k],
                    recv_sem=ag_rR_sem.at[k],
                    device_id=(right,), device_id_type=MESH,
                )
                rdma.start()
                agR[k] = rdma
            if t == 0:
                ag_direct = pltpu.make_async_remote_copy(
                    src_ref=out_ref.at[p],
                    dst_ref=out_ref.at[p],
                    send_sem=ag_sD_sem, recv_sem=ag_rD_sem,
                    device_id=(anti,), device_id_type=MESH,
                )
                ag_direct.start()

        for j in range(SUB):
            agL[(STEPS - 1) * SUB + j].wait()
            agR[(STEPS - 1) * SUB + j].wait()
        ag_direct.wait()

    out = pl.pallas_call(
        body,
        out_shape=jax.ShapeDtypeStruct((N_DEV, CH, N), jnp.bfloat16),
        in_specs=[pl.BlockSpec(memory_space=pltpu.VMEM)],
        out_specs=pl.BlockSpec(memory_space=pltpu.VMEM),
        scratch_shapes=[
            pltpu.VMEM((STEPS, CH, N), jnp.bfloat16),
            pltpu.VMEM((STEPS, CH, N), jnp.bfloat16),
            pltpu.VMEM((STEPS, CH, N), jnp.bfloat16),
            pltpu.VMEM((STEPS, CH, N), jnp.bfloat16),
            pltpu.VMEM((CH, N), jnp.bfloat16),
            pltpu.VMEM((CH, N), jnp.bfloat16),
            pltpu.SemaphoreType.DMA((STEPS * SUB,)),
            pltpu.SemaphoreType.DMA((STEPS * SUB,)),
            pltpu.SemaphoreType.DMA((STEPS * SUB,)),
            pltpu.SemaphoreType.DMA((STEPS * SUB,)),
            pltpu.SemaphoreType.DMA((STEPS * SUB,)),
            pltpu.SemaphoreType.DMA((STEPS * SUB,)),
            pltpu.SemaphoreType.DMA((STEPS * SUB,)),
            pltpu.SemaphoreType.DMA((STEPS * SUB,)),
            pltpu.SemaphoreType.DMA(()),
            pltpu.SemaphoreType.DMA(()),
            pltpu.SemaphoreType.DMA(()),
            pltpu.SemaphoreType.DMA(()),
        ],
        compiler_params=pltpu.CompilerParams(collective_id=0),
    )(x)
    return out.reshape(M, N)


# device time: 66366 ns/iter; 1.0647x vs baseline; 1.0642x over previous
import jax
import jax.numpy as jnp
from jax import lax
from jax.experimental import pallas as pl
from jax.experimental.pallas import tpu as pltpu

N_DEV = 16
M, N = 2048, 1024
CH = M // N_DEV
SUB = 8
CS = CH // SUB
L_STEPS = 8
R_STEPS = 7
MESH = pl.DeviceIdType.MESH


def kernel(x):
    def body(x_ref, out_ref, sL, sR, rL, rR,
             rs_sL_sem, rs_rL_sem, rs_sR_sem, rs_rR_sem,
             ag_sL_sem, ag_rL_sem, ag_sR_sem, ag_rR_sem):
        p = lax.axis_index("i")
        left = (p + N_DEV - 1) % N_DEV
        right = (p + 1) % N_DEV

        def xchunk(c, j):
            sl = pl.ds(c * CH + j * CS, CS)
            return x_ref[0, sl, :].astype(jnp.bfloat16)

        barrier = pltpu.get_barrier_semaphore()
        for nbr in (left, right):
            pl.semaphore_signal(
                barrier, inc=1, device_id=(nbr,), device_id_type=MESH,
            )
        pl.semaphore_wait(barrier, 2)

        rsL = [None] * (L_STEPS * SUB)
        rsR = [None] * (R_STEPS * SUB)
        for s in range(L_STEPS):
            cL = (p + L_STEPS + s) % N_DEV
            cR = (p + R_STEPS - s) % N_DEV
            for j in range(SUB):
                rows = slice(j * CS, (j + 1) * CS)
                k = s * SUB + j
                if s > 0:
                    rsL[k - SUB].wait()
                    sL[s, rows, :] = xchunk(cL, j) + rL[s - 1, rows, :]
                else:
                    sL[s, rows, :] = xchunk(cL, j)
                rdma = pltpu.make_async_remote_copy(
                    src_ref=sL.at[s, rows],
                    dst_ref=rL.at[s, rows],
                    send_sem=rs_sL_sem.at[k],
                    recv_sem=rs_rL_sem.at[k],
                    device_id=(left,), device_id_type=MESH,
                )
                rdma.start()
                rsL[k] = rdma
                if s < R_STEPS:
                    if s > 0:
                        rsR[k - SUB].wait()
                        sR[s, rows, :] = xchunk(cR, j) + rR[s - 1, rows, :]
                    else:
                        sR[s, rows, :] = xchunk(cR, j)
                    rdma = pltpu.make_async_remote_copy(
                        src_ref=sR.at[s, rows],
                        dst_ref=rR.at[s, rows],
                        send_sem=rs_sR_sem.at[k],
                        recv_sem=rs_rR_sem.at[k],
                        device_id=(right,), device_id_type=MESH,
                    )
                    rdma.start()
                    rsR[k] = rdma

        agL = [None] * (L_STEPS * SUB)
        agR = [None] * (R_STEPS * SUB)
        for t in range(L_STEPS):
            cL = (p + t) % N_DEV
            cR = (p + N_DEV - t) % N_DEV
            for j in range(SUB):
                rows = slice(j * CS, (j + 1) * CS)
                k = t * SUB + j
                if t > 0:
                    agL[k - SUB].wait()
                else:
                    rsL[(L_STEPS - 1) * SUB + j].wait()
                    rsR[(R_STEPS - 1) * SUB + j].wait()
                    out_ref[p, rows, :] = (
                        xchunk(p, j)
                        + rL[L_STEPS - 1, rows, :]
                        + rR[R_STEPS - 1, rows, :]
                    )
                rdma = pltpu.make_async_remote_copy(
                    src_ref=out_ref.at[cL, rows],
                    dst_ref=out_ref.at[cL, rows],
                    send_sem=ag_sL_sem.at[k],
                    recv_sem=ag_rL_sem.at[k],
                    device_id=(left,), device_id_type=MESH,
                )
                rdma.start()
                agL[k] = rdma
                if t < R_STEPS:
                    if t > 0:
                        agR[k - SUB].wait()
                    rdma = pltpu.make_async_remote_copy(
                        src_ref=out_ref.at[cR, rows],
                        dst_ref=out_ref.at[cR, rows],
                        send_sem=ag_sR_sem.at[k],
                        recv_sem=ag_rR_sem.at[k],
                        device_id=(right,), device_id_type=MESH,
                    )
                    rdma.start()
                    agR[k] = rdma

        for j in range(SUB):
            agL[(L_STEPS - 1) * SUB + j].wait()
            agR[(R_STEPS - 1) * SUB + j].wait()

    out = pl.pallas_call(
        body,
        out_shape=jax.ShapeDtypeStruct((N_DEV, CH, N), jnp.bfloat16),
        in_specs=[pl.BlockSpec(memory_space=pltpu.VMEM)],
        out_specs=pl.BlockSpec(memory_space=pltpu.VMEM),
        scratch_shapes=[
            pltpu.VMEM((L_STEPS, CH, N), jnp.bfloat16),
            pltpu.VMEM((R_STEPS, CH, N), jnp.bfloat16),
            pltpu.VMEM((L_STEPS, CH, N), jnp.bfloat16),
            pltpu.VMEM((R_STEPS, CH, N), jnp.bfloat16),
            pltpu.SemaphoreType.DMA((L_STEPS * SUB,)),
            pltpu.SemaphoreType.DMA((L_STEPS * SUB,)),
            pltpu.SemaphoreType.DMA((R_STEPS * SUB,)),
            pltpu.SemaphoreType.DMA((R_STEPS * SUB,)),
            pltpu.SemaphoreType.DMA((L_STEPS * SUB,)),
            pltpu.SemaphoreType.DMA((L_STEPS * SUB,)),
            pltpu.SemaphoreType.DMA((R_STEPS * SUB,)),
            pltpu.SemaphoreType.DMA((R_STEPS * SUB,)),
        ],
        compiler_params=pltpu.CompilerParams(collective_id=0),
    )(x)
    return out.reshape(M, N)
